# baseline (speedup 1.0000x reference)
import jax
import jax.numpy as jnp
from jax import lax
from jax.experimental import pallas as pl
from jax.experimental.pallas import tpu as pltpu
from jax.experimental.pallas import tpu_sc as plsc

VOCAB = 1000
EMB_DIM = 4
B, L = 4096, 50

NC, NS = 2, 16
NW = NC * NS
BT = B // NW
VT = VOCAB // 8
NP = 5
PVT = VT // NP
COLS = PVT * 8
LANES = 16


def _table_body(emb_ref, w_ref, b_ref, out_ref):
    acc = jax.lax.dot_general(
        emb_ref[...], w_ref[0],
        (((1,), (1,)), ((), ())),
        preferred_element_type=jnp.float32,
    )
    out_ref[0] = acc + b_ref[0]


def _make_table(emb_table, W, b):
    return pl.pallas_call(
        _table_body,
        grid=(NP,),
        in_specs=[
            pl.BlockSpec((VOCAB, EMB_DIM), lambda p: (0, 0)),
            pl.BlockSpec((1, COLS, EMB_DIM), lambda p: (p, 0, 0)),
            pl.BlockSpec((1, 1, COLS), lambda p: (p, 0, 0)),
        ],
        out_specs=pl.BlockSpec((1, VOCAB, COLS), lambda p: (p, 0, 0)),
        out_shape=jax.ShapeDtypeStruct((NP, VOCAB, COLS), jnp.float32),
    )(emb_table, W.reshape(NP, COLS, EMB_DIM), b.reshape(NP, 1, COLS))


def _sc_gather_body(table_hbm, xt_hbm, out_hbm,
                    idx_v, rows0, rows1, tr0, tr1,
                    gs0, gs1, ws0, ws1):
    c = lax.axis_index("c")
    s = lax.axis_index("s")
    wid = s * NC + c

    iota = lax.iota(jnp.int32, LANES)
    bvecs = [iota + b2 for b2 in range(0, BT, LANES)]

    pltpu.sync_copy(xt_hbm.at[pl.ds(0, L), pl.ds(wid * BT, BT)], idx_v)

    NCH = L * NP

    def gather(C, rows_v, sem):
        l = C // NP
        p = lax.rem(C, NP)
        src = table_hbm.at[p].at[idx_v.at[l]]
        return pltpu.make_async_copy(src, rows_v, sem)

    def out_dma(C, tr_v, sem):
        l = C // NP
        p = lax.rem(C, NP)
        dst = out_hbm.at[l, pl.ds(p * PVT, PVT), wid]
        return pltpu.make_async_copy(tr_v, dst, sem)

    def transpose(rows_v, tr_v):
        @plsc.parallel_loop(0, PVT, unroll=5)
        def _(vt):
            for vi in range(8):
                col_v = jnp.full((LANES,), vt * 8 + vi, jnp.int32)
                for k, bv in enumerate(bvecs):
                    tr_v[vt, vi, pl.ds(k * LANES, LANES)] = (
                        plsc.load_gather(rows_v, [bv, col_v]))

    bufs = [(rows0, gs0, tr0, ws0), (rows1, gs1, tr1, ws1)]
    NDEEP = len(bufs)

    for k in range(NDEEP):
        gather(k, bufs[k][0], bufs[k][1]).start()

    def step(t2, carry):
        for k in range(NDEEP):
            C = t2 * NDEEP + k
            rows_v, gsem, tr_v, wsem = bufs[k]
            gather(C, rows_v, gsem).wait()

            @pl.when(C >= NDEEP)
            def _():
                out_dma(C, tr_v, wsem).wait()

            transpose(rows_v, tr_v)
            out_dma(C, tr_v, wsem).start()

            @pl.when(C + NDEEP < NCH)
            def _():
                gather(C + NDEEP, rows_v, gsem).start()
        return carry

    lax.fori_loop(0, NCH // NDEEP, step, 0)
    for C in range(NCH - NDEEP, NCH):
        _, _, tr_v, wsem = bufs[C % NDEEP]
        out_dma(C, tr_v, wsem).wait()


def _gather_rows(table, x_t):
    mesh = plsc.VectorSubcoreMesh(core_axis_name="c", subcore_axis_name="s")
    return pl.kernel(
        _sc_gather_body,
        out_type=jax.ShapeDtypeStruct((L, VT, NW, 8, BT), jnp.float32),
        mesh=mesh,
        scratch_types=[
            pltpu.VMEM((L, BT), jnp.int32),
            pltpu.VMEM((BT, COLS), jnp.float32),
            pltpu.VMEM((BT, COLS), jnp.float32),
            pltpu.VMEM((PVT, 8, BT), jnp.float32),
            pltpu.VMEM((PVT, 8, BT), jnp.float32),
            pltpu.SemaphoreType.DMA,
            pltpu.SemaphoreType.DMA,
            pltpu.SemaphoreType.DMA,
            pltpu.SemaphoreType.DMA,
        ],
        compiler_params=pltpu.CompilerParams(
            use_tc_tiling_on_sc=False, needs_layout_passes=False
        ),
    )(table, x_t)


def kernel(x, emb_table, W, b):
    table = _make_table(emb_table, W, b)
    x_t = x.astype(jnp.int32).T
    out5 = _gather_rows(table, x_t)
    return out5.transpose(2, 4, 0, 1, 3).reshape(B, L, VOCAB)


# speedup vs baseline: 1.1131x; 1.1131x over previous
import jax
import jax.numpy as jnp
from jax import lax
from jax.experimental import pallas as pl
from jax.experimental.pallas import tpu as pltpu
from jax.experimental.pallas import tpu_sc as plsc

VOCAB = 1000
EMB_DIM = 4
B, L = 4096, 50

NC, NS = 2, 16
NW = NC * NS
BT = B // NW
VT = VOCAB // 8
QW = 16
NQ = BT // QW
LANES = 16


def _table_body(emb_ref, w_ref, b_ref, out_ref):
    acc = jax.lax.dot_general(
        emb_ref[...], w_ref[...],
        (((1,), (1,)), ((), ())),
        preferred_element_type=jnp.float32,
    )
    out_ref[...] = acc + b_ref[...]


def _make_table(emb_table, W, b):
    return pl.pallas_call(
        _table_body,
        out_shape=jax.ShapeDtypeStruct((VOCAB, VOCAB), jnp.float32),
    )(emb_table, W, b.reshape(1, VOCAB))


def _sc_gather_body(table_hbm, xt_hbm, out_hbm,
                    idx_v, rows0, rows1, rows2, tr0, tr1, tr2,
                    gs0, gs1, gs2, ws0, ws1, ws2):
    c = lax.axis_index("c")
    s = lax.axis_index("s")
    wid = s * NC + c

    iota = lax.iota(jnp.int32, LANES)
    bvecs = [iota + b2 for b2 in range(0, QW, LANES)]

    pltpu.sync_copy(xt_hbm.at[pl.ds(0, L), pl.ds(wid * BT, BT)], idx_v)

    NQTOT = L * NQ

    def gather(Q, rows_v, sem):
        l = Q // NQ
        q = lax.rem(Q, NQ)
        src = table_hbm.at[idx_v.at[l, pl.ds(q * QW, QW)]]
        return pltpu.make_async_copy(src, rows_v, sem)

    def out_dma(Q, tr_v, sem):
        l = Q // NQ
        q = lax.rem(Q, NQ)
        dst = out_hbm.at[l, pl.ds(0, VT), wid, pl.ds(0, 8), pl.ds(q * QW, QW)]
        return pltpu.make_async_copy(tr_v, dst, sem)

    def transpose(rows_v, tr_v):
        @plsc.parallel_loop(0, VT, unroll=4)
        def _(vt):
            for vi in range(8):
                col_v = jnp.full((LANES,), vt * 8 + vi, jnp.int32)
                for k, bv in enumerate(bvecs):
                    tr_v[vt, vi, pl.ds(k * LANES, LANES)] = (
                        plsc.load_gather(rows_v, [bv, col_v]))

    bufs = [(rows0, gs0, tr0, ws0), (rows1, gs1, tr1, ws1),
            (rows2, gs2, tr2, ws2)]
    NDEEP = len(bufs)

    for k in range(NDEEP):
        gather(k, bufs[k][0], bufs[k][1]).start()

    NTRIPLE = -(-NQTOT // NDEEP)

    def triple(t3, carry):
        for k in range(NDEEP):
            Q = t3 * NDEEP + k
            rows_v, gsem, tr_v, wsem = bufs[k]

            @pl.when(Q < NQTOT)
            def _():
                gather(Q, rows_v, gsem).wait()

                @pl.when(Q >= NDEEP)
                def _():
                    out_dma(Q, tr_v, wsem).wait()

                transpose(rows_v, tr_v)
                out_dma(Q, tr_v, wsem).start()

                @pl.when(Q + NDEEP < NQTOT)
                def _():
                    gather(Q + NDEEP, rows_v, gsem).start()
        return carry

    lax.fori_loop(0, NTRIPLE, triple, 0)
    for Q in range(NQTOT - NDEEP, NQTOT):
        _, _, tr_v, wsem = bufs[Q % NDEEP]
        out_dma(Q, tr_v, wsem).wait()


def _gather_rows(table, x_t):
    mesh = plsc.VectorSubcoreMesh(core_axis_name="c", subcore_axis_name="s")
    return pl.kernel(
        _sc_gather_body,
        out_type=jax.ShapeDtypeStruct((L, VT, NW, 8, BT), jnp.float32),
        mesh=mesh,
        scratch_types=[
            pltpu.VMEM((L, BT), jnp.int32),
            pltpu.VMEM((QW, VOCAB), jnp.float32),
            pltpu.VMEM((QW, VOCAB), jnp.float32),
            pltpu.VMEM((QW, VOCAB), jnp.float32),
            pltpu.VMEM((VT, 8, QW), jnp.float32),
            pltpu.VMEM((VT, 8, QW), jnp.float32),
            pltpu.VMEM((VT, 8, QW), jnp.float32),
            pltpu.SemaphoreType.DMA,
            pltpu.SemaphoreType.DMA,
            pltpu.SemaphoreType.DMA,
            pltpu.SemaphoreType.DMA,
            pltpu.SemaphoreType.DMA,
            pltpu.SemaphoreType.DMA,
        ],
        compiler_params=pltpu.CompilerParams(
            use_tc_tiling_on_sc=False, needs_layout_passes=False
        ),
    )(table, x_t)


def kernel(x, emb_table, W, b):
    table = _make_table(emb_table, W, b)
    x_t = x.astype(jnp.int32).T
    out5 = _gather_rows(table, x_t)
    return out5.transpose(2, 4, 0, 1, 3).reshape(B, L, VOCAB)
